# baseline (speedup 1.0000x reference)
import functools

import jax
import jax.numpy as jnp
from jax import lax
from jax.experimental import pallas as pl
from jax.experimental.pallas import tpu as pltpu
from jax.experimental.pallas import tpu_sc as plsc

_B = 20
_NPG = 500
_NP = 512
_F = 128
_H = 128
_K = 20
_KP = 32
_M1 = 400
_M2 = 320
_BIG = 1e30
_JBIG = 2**30


def _dotbf(a, b):
    return lax.dot_general(a.astype(jnp.bfloat16), b.astype(jnp.bfloat16),
                           (((1,), (0,)), ((), ())),
                           preferred_element_type=jnp.float32)


def _dotbf_t(a, b):
    return lax.dot_general(a.astype(jnp.bfloat16), b.astype(jnp.bfloat16),
                           (((1,), (1,)), ((), ())),
                           preferred_element_type=jnp.float32)


def _dot_hi(a, b):
    return lax.dot_general(a, b, (((1,), (0,)), ((), ())),
                           precision=lax.Precision.HIGHEST,
                           preferred_element_type=jnp.float32)


def _dot_hi_t(a, b):
    return lax.dot_general(a, b, (((1,), (1,)), ((), ())),
                           precision=lax.Precision.HIGHEST,
                           preferred_element_type=jnp.float32)


def _pairwise_d2(xp, colpen_row):
    xx = xp * xp
    sq_col = jnp.sum(xx, axis=1, keepdims=True)
    sq_row = _dot_hi_t(jnp.ones((1, _F), jnp.float32), xx)
    mm = _dotbf_t(xp, xp)
    return (sq_col + sq_row) - 2.0 * mm + colpen_row


def _row_of_col(col):
    sub = lax.broadcasted_iota(jnp.int32, (_NP, _NP), 0)
    lan = lax.broadcasted_iota(jnp.int32, (_NP, _NP), 1)
    diag = (sub == lan).astype(jnp.float32) * col
    return _dot_hi(jnp.ones((1, _NP), jnp.float32), diag)


def _topk_idx(d2_ref, keep_i32):
    lane = lax.broadcasted_iota(jnp.int32, (_NP, _NP), 1)
    klane = lax.broadcasted_iota(jnp.int32, (_NP, _KP), 1)

    def body(k, idx_acc):
        d2 = d2_ref[...]
        m = jnp.min(d2, axis=1, keepdims=True)
        cand = jnp.where(d2 == m, lane, _JBIG)
        j = jnp.min(cand, axis=1, keepdims=True)
        idx_acc = jnp.where(klane == k, j, idx_acc)
        d2_ref[...] = jnp.where(lane == j, _BIG, d2)
        return idx_acc

    idx_acc = lax.fori_loop(0, _K, body, jnp.zeros((_NP, _KP), jnp.int32))
    return jnp.where(klane == _K, keep_i32, idx_acc)


def _select_top(svalid, m):
    bits = lax.bitcast_convert_type(svalid, jnp.int32)
    key = jnp.where(bits < 0, bits ^ jnp.int32(0x7FFFFFFF), bits)
    ukey = key ^ jnp.int32(-(2**31))
    hi = lax.shift_right_logical(ukey, jnp.int32(16)).astype(jnp.float32)
    lo = (ukey & jnp.int32(0xFFFF)).astype(jnp.float32)
    hi_row = _row_of_col(hi)
    lo_row = _row_of_col(lo)
    beats = (hi_row > hi) | ((hi_row == hi) & (lo_row > lo))
    sub = lax.broadcasted_iota(jnp.int32, (_NP, _NP), 0)
    lan = lax.broadcasted_iota(jnp.int32, (_NP, _NP), 1)
    tie_before = (hi_row == hi) & (lo_row == lo) & (lan < sub)
    rank = jnp.sum(beats.astype(jnp.float32) + tie_before.astype(jnp.float32),
                   axis=1, keepdims=True)
    return rank < jnp.float32(m)


def _gap(hp, wg_row, keepf):
    gate = jnp.sum(hp * wg_row, axis=1, keepdims=True)
    gate = jnp.where(keepf > 0.0, gate, -_BIG)
    e = jnp.exp(gate - jnp.max(gate)) * keepf
    w = e / jnp.sum(e)
    return jnp.sum(w * hp, axis=0, keepdims=True)


def _conv_max(xi, xj_ref, wa_ref, ba_row, wb_ref, bb_row, hm_ref):
    hm_ref[...] = jnp.full((_NP, _H), -_BIG, jnp.float32)

    def body(k, carry):
        xj = xj_ref[0, pl.ds(k * _NP, _NP), :]
        z = jnp.concatenate([xi, xj - xi], axis=1)
        msg = _dotbf(jnp.maximum(_dotbf(z, wa_ref[...]) + ba_row, 0.0),
                     wb_ref[...]) + bb_row
        hm_ref[...] = jnp.maximum(hm_ref[...], msg)
        return carry

    lax.fori_loop(0, _K, body, 0)
    return hm_ref[...]


def _stage_a_body(x_ref, idx_ref, d2_ref):
    g = pl.program_id(0)
    x = x_ref[0]
    col = lax.broadcasted_iota(jnp.int32, (1, _NP), 1)
    colpen_row = jnp.where(col >= _NPG, _BIG, 0.0)
    d2_ref[...] = _pairwise_d2(x, colpen_row)
    idx_acc = _topk_idx(d2_ref, jnp.zeros((_NP, 1), jnp.int32))
    klane = lax.broadcasted_iota(jnp.int32, (_NP, _KP), 1)
    idx_ref[0] = jnp.where(klane < _K, idx_acc + g * _NP, idx_acc)


def _run_stage_a(xp):
    nb = xp.shape[0]
    return pl.pallas_call(
        _stage_a_body,
        grid=(nb,),
        in_specs=[pl.BlockSpec((1, _NP, _F), lambda g: (g, 0, 0))],
        out_specs=[pl.BlockSpec((1, _NP, _KP), lambda g: (g, 0, 0))],
        out_shape=[jax.ShapeDtypeStruct((nb, _NP, _KP), jnp.int32)],
        scratch_shapes=[pltpu.VMEM((_NP, _NP), jnp.float32)],
    )(xp)


def _stage_c_body(x_ref, xj_ref, w1a_ref, b1a_ref, w1b_ref, b1b_ref, p1_ref,
                  wg1_ref, hp_ref, idx2_ref, x1_ref, d2_ref, hm_ref):
    g = pl.program_id(0)
    h = jnp.maximum(_conv_max(x_ref[0], xj_ref, w1a_ref, b1a_ref[...],
                              w1b_ref, b1b_ref[...], hm_ref), 0.0)
    p1 = p1_ref[...]
    s = jnp.sum(h * p1, axis=1, keepdims=True) / (jnp.sqrt(jnp.sum(p1 * p1)) + 1e-16)
    row = lax.broadcasted_iota(jnp.int32, (_NP, 1), 0)
    keep = _select_top(jnp.where(row < _NPG, s, -_BIG), _M1)
    keepf = keep.astype(jnp.float32)
    hp = h * jnp.tanh(s) * keepf
    hp_ref[0] = hp
    x1_ref[0] = _gap(hp, wg1_ref[...], keepf)
    colpen_row = (1.0 - _row_of_col(keepf)) * _BIG
    d2_ref[...] = _pairwise_d2(hp, colpen_row)
    idx_acc = _topk_idx(d2_ref, keep.astype(jnp.int32))
    klane = lax.broadcasted_iota(jnp.int32, (_NP, _KP), 1)
    idx2_ref[0] = jnp.where(klane < _K, idx_acc + g * _NP, idx_acc)


def _run_stage_c(xp, xj1, W1a, b1a_row, W1b, b1b_row, p1_row, wg1_row):
    nb = xp.shape[0]
    return pl.pallas_call(
        _stage_c_body,
        grid=(nb,),
        in_specs=[
            pl.BlockSpec((1, _NP, _F), lambda g: (g, 0, 0)),
            pl.BlockSpec((1, _K * _NP, _F), lambda g: (g, 0, 0)),
            pl.BlockSpec((2 * _F, _H), lambda g: (0, 0)),
            pl.BlockSpec((1, _H), lambda g: (0, 0)),
            pl.BlockSpec((_H, _H), lambda g: (0, 0)),
            pl.BlockSpec((1, _H), lambda g: (0, 0)),
            pl.BlockSpec((1, _H), lambda g: (0, 0)),
            pl.BlockSpec((1, _H), lambda g: (0, 0)),
        ],
        out_specs=[
            pl.BlockSpec((1, _NP, _H), lambda g: (g, 0, 0)),
            pl.BlockSpec((1, _NP, _KP), lambda g: (g, 0, 0)),
            pl.BlockSpec((1, 1, _H), lambda g: (g, 0, 0)),
        ],
        out_shape=[
            jax.ShapeDtypeStruct((nb, _NP, _H), jnp.float32),
            jax.ShapeDtypeStruct((nb, _NP, _KP), jnp.int32),
            jax.ShapeDtypeStruct((nb, 1, _H), jnp.float32),
        ],
        scratch_shapes=[pltpu.VMEM((_NP, _NP), jnp.float32),
                        pltpu.VMEM((_NP, _H), jnp.float32)],
    )(xp, xj1, W1a, b1a_row, W1b, b1b_row, p1_row, wg1_row)


def _stage_e_body(hp_ref, hj_ref, idx2_ref, w2a_ref, b2a_ref, w2b_ref, b2b_ref,
                  p2_ref, wg2_ref, wl_ref, bl_ref, x1_ref, out_ref, hm_ref):
    h2 = _conv_max(hp_ref[0], hj_ref, w2a_ref, b2a_ref[...],
                   w2b_ref, b2b_ref[...], hm_ref)
    keep1 = idx2_ref[0][:, _K:_K + 1] > 0
    p2 = p2_ref[...]
    s2 = jnp.sum(h2 * p2, axis=1, keepdims=True) / (jnp.sqrt(jnp.sum(p2 * p2)) + 1e-16)
    keep2 = _select_top(jnp.where(keep1, s2, -_BIG), _M2)
    keep2f = keep2.astype(jnp.float32)
    hp2 = h2 * jnp.tanh(s2) * keep2f
    x2 = _gap(hp2, wg2_ref[...], keep2f)
    out_ref[0] = _dotbf(x1_ref[0] + x2, wl_ref[...]) + bl_ref[...]


def _run_stage_e(hp, hj2, idx2, W2a, b2a_row, W2b, b2b_row, p2_row, wg2_row,
                 wlp, blp_row, x1):
    nb = hp.shape[0]
    return pl.pallas_call(
        _stage_e_body,
        grid=(nb,),
        in_specs=[
            pl.BlockSpec((1, _NP, _H), lambda g: (g, 0, 0)),
            pl.BlockSpec((1, _K * _NP, _H), lambda g: (g, 0, 0)),
            pl.BlockSpec((1, _NP, _KP), lambda g: (g, 0, 0)),
            pl.BlockSpec((2 * _H, _H), lambda g: (0, 0)),
            pl.BlockSpec((1, _H), lambda g: (0, 0)),
            pl.BlockSpec((_H, _H), lambda g: (0, 0)),
            pl.BlockSpec((1, _H), lambda g: (0, 0)),
            pl.BlockSpec((1, _H), lambda g: (0, 0)),
            pl.BlockSpec((1, _H), lambda g: (0, 0)),
            pl.BlockSpec((_H, _H), lambda g: (0, 0)),
            pl.BlockSpec((1, _H), lambda g: (0, 0)),
            pl.BlockSpec((1, 1, _H), lambda g: (g, 0, 0)),
        ],
        out_specs=[pl.BlockSpec((1, 1, _H), lambda g: (g, 0, 0))],
        out_shape=[jax.ShapeDtypeStruct((nb, 1, _H), jnp.float32)],
        scratch_shapes=[pltpu.VMEM((_NP, _H), jnp.float32)],
    )(hp, hj2, idx2, W2a, b2a_row, W2b, b2b_row, p2_row, wg2_row, wlp, blp_row, x1)


def _sc_gather_rows(table, idx):
    e = idx.shape[0]
    info = plsc.get_sparse_core_info()
    nw = info.num_cores * info.num_subcores
    per_w = e // nw
    nch = -(-per_w // 800)
    while per_w % nch or (per_w // nch) % 8:
        nch += 1
    ch = per_w // nch
    mesh = plsc.VectorSubcoreMesh(core_axis_name="c", subcore_axis_name="s")

    @functools.partial(
        pl.kernel, mesh=mesh,
        out_type=jax.ShapeDtypeStruct((e, _H), jnp.float32),
        scratch_types=[pltpu.VMEM((ch,), jnp.int32),
                       pltpu.VMEM((ch, _H), jnp.float32),
                       pltpu.SemaphoreType.DMA],
    )
    def gk(table_hbm, idx_hbm, out_hbm, idx_v, rows_v, sem):
        wid = lax.axis_index("s") * info.num_cores + lax.axis_index("c")
        base = wid * per_w

        def chunk(c, carry):
            off = pl.multiple_of(base + c * ch, 8)
            pltpu.sync_copy(idx_hbm.at[pl.ds(off, ch)], idx_v)
            pltpu.async_copy(table_hbm.at[idx_v], rows_v, sem).wait()
            pltpu.sync_copy(rows_v, out_hbm.at[pl.ds(off, ch)])
            return carry

        lax.fori_loop(0, nch, chunk, 0)

    return gk(table, idx)


def kernel(x, edge_index, edge_weight, batch, W1a, b1a, W1b, b1b, p1, Wg1, bg1,
           W2a, b2a, W2b, b2b, p2, Wg2, bg2, Wl, bl):
    xp = jnp.pad(x.reshape(_B, _NPG, _F), ((0, 0), (0, _NP - _NPG), (0, 0)))
    wlp = jnp.concatenate([Wl, jnp.zeros((_H, _H - Wl.shape[1]), jnp.float32)], axis=1)
    blp = jnp.concatenate([bl, jnp.zeros((_H - bl.shape[0],), jnp.float32)]).reshape(1, _H)

    hb = _B // 4
    outs = []
    for xph in (xp[:hb], xp[hb:2 * hb], xp[2 * hb:3 * hb], xp[3 * hb:]):
        idx1 = _run_stage_a(xph)[0]
        idx1_flat = jnp.transpose(idx1[:, :, :_K], (0, 2, 1)).reshape(-1)
        xj1 = _sc_gather_rows(xph.reshape(hb * _NP, _F), idx1_flat)
        hp, idx2, x1 = _run_stage_c(
            xph, xj1.reshape(hb, _K * _NP, _F), W1a, b1a.reshape(1, _H), W1b,
            b1b.reshape(1, _H), p1.reshape(1, _H), Wg1.reshape(1, _H))
        idx2_flat = jnp.transpose(idx2[:, :, :_K], (0, 2, 1)).reshape(-1)
        hj2 = _sc_gather_rows(hp.reshape(hb * _NP, _H), idx2_flat)
        out = _run_stage_e(hp, hj2.reshape(hb, _K * _NP, _H), idx2, W2a,
                           b2a.reshape(1, _H), W2b, b2b.reshape(1, _H),
                           p2.reshape(1, _H), Wg2.reshape(1, _H), wlp, blp, x1)[0]
        outs.append(out[:, 0, :Wl.shape[1]])
    return jnp.concatenate(outs, axis=0)


# speedup vs baseline: 1.0109x; 1.0109x over previous
import functools

import jax
import jax.numpy as jnp
from jax import lax
from jax.experimental import pallas as pl
from jax.experimental.pallas import tpu as pltpu
from jax.experimental.pallas import tpu_sc as plsc

_B = 20
_NPG = 500
_NP = 512
_F = 128
_H = 128
_K = 20
_KP = 32
_M1 = 400
_M2 = 320
_BIG = 1e30
_JBIG = 2**30


def _dotbf(a, b):
    return lax.dot_general(a.astype(jnp.bfloat16), b.astype(jnp.bfloat16),
                           (((1,), (0,)), ((), ())),
                           preferred_element_type=jnp.float32)


def _dotbf_t(a, b):
    return lax.dot_general(a.astype(jnp.bfloat16), b.astype(jnp.bfloat16),
                           (((1,), (1,)), ((), ())),
                           preferred_element_type=jnp.float32)


def _dot_hi(a, b):
    return lax.dot_general(a, b, (((1,), (0,)), ((), ())),
                           precision=lax.Precision.HIGHEST,
                           preferred_element_type=jnp.float32)


def _dot_hi_t(a, b):
    return lax.dot_general(a, b, (((1,), (1,)), ((), ())),
                           precision=lax.Precision.HIGHEST,
                           preferred_element_type=jnp.float32)


def _pairwise_d2(xp, colpen_row):
    xx = xp * xp
    sq_col = jnp.sum(xx, axis=1, keepdims=True)
    sq_row = _dot_hi_t(jnp.ones((1, _F), jnp.float32), xx)
    mm = _dotbf_t(xp, xp)
    return (sq_col + sq_row) - 2.0 * mm + colpen_row


def _row_of_col(col):
    sub = lax.broadcasted_iota(jnp.int32, (_NP, _NP), 0)
    lan = lax.broadcasted_iota(jnp.int32, (_NP, _NP), 1)
    diag = (sub == lan).astype(jnp.float32) * col
    return _dot_hi(jnp.ones((1, _NP), jnp.float32), diag)


def _topk_idx(d2_ref, keep_i32):
    lane = lax.broadcasted_iota(jnp.int32, (_NP, _NP), 1)
    klane = lax.broadcasted_iota(jnp.int32, (_NP, _KP), 1)

    def body(k, idx_acc):
        d2 = d2_ref[...]
        m = jnp.min(d2, axis=1, keepdims=True)
        cand = jnp.where(d2 == m, lane, _JBIG)
        j = jnp.min(cand, axis=1, keepdims=True)
        idx_acc = jnp.where(klane == k, j, idx_acc)
        d2_ref[...] = jnp.where(lane == j, _BIG, d2)
        return idx_acc

    idx_acc = lax.fori_loop(0, _K, body, jnp.zeros((_NP, _KP), jnp.int32))
    return jnp.where(klane == _K, keep_i32, idx_acc)


def _select_top(svalid, m):
    bits = lax.bitcast_convert_type(svalid, jnp.int32)
    key = jnp.where(bits < 0, bits ^ jnp.int32(0x7FFFFFFF), bits)
    ukey = key ^ jnp.int32(-(2**31))
    hi = lax.shift_right_logical(ukey, jnp.int32(16)).astype(jnp.float32)
    lo = (ukey & jnp.int32(0xFFFF)).astype(jnp.float32)
    hi_row = _row_of_col(hi)
    lo_row = _row_of_col(lo)
    beats = (hi_row > hi) | ((hi_row == hi) & (lo_row > lo))
    sub = lax.broadcasted_iota(jnp.int32, (_NP, _NP), 0)
    lan = lax.broadcasted_iota(jnp.int32, (_NP, _NP), 1)
    tie_before = (hi_row == hi) & (lo_row == lo) & (lan < sub)
    rank = jnp.sum(beats.astype(jnp.float32) + tie_before.astype(jnp.float32),
                   axis=1, keepdims=True)
    return rank < jnp.float32(m)


def _gap(hp, wg_row, keepf):
    gate = _dotbf_t(hp, wg_row)[:, :1]
    gate = jnp.where(keepf > 0.0, gate, -_BIG)
    e = jnp.exp(gate - jnp.max(gate)) * keepf
    w = e / jnp.sum(e)
    return jnp.sum(w * hp, axis=0, keepdims=True)


def _conv_max(xi, xj_ref, wa_ref, ba_row, wb_ref, bb_row, hm_ref):
    hm_ref[...] = jnp.full((_NP, _H), -_BIG, jnp.float32)

    def body(k, carry):
        xj = xj_ref[0, pl.ds(k * _NP, _NP), :]
        z = jnp.concatenate([xi, xj - xi], axis=1)
        msg = _dotbf(jnp.maximum(_dotbf(z, wa_ref[...]) + ba_row, 0.0),
                     wb_ref[...]) + bb_row
        hm_ref[...] = jnp.maximum(hm_ref[...], msg)
        return carry

    lax.fori_loop(0, _K, body, 0)
    return hm_ref[...]


def _stage_a_body(x_ref, idx_ref, d2_ref):
    g = pl.program_id(0)
    x = x_ref[0]
    col = lax.broadcasted_iota(jnp.int32, (1, _NP), 1)
    colpen_row = jnp.where(col >= _NPG, _BIG, 0.0)
    d2_ref[...] = _pairwise_d2(x, colpen_row)
    idx_acc = _topk_idx(d2_ref, jnp.zeros((_NP, 1), jnp.int32))
    klane = lax.broadcasted_iota(jnp.int32, (_NP, _KP), 1)
    idx_ref[0] = jnp.where(klane < _K, idx_acc + g * _NP, idx_acc)


def _run_stage_a(xp):
    nb = xp.shape[0]
    return pl.pallas_call(
        _stage_a_body,
        grid=(nb,),
        in_specs=[pl.BlockSpec((1, _NP, _F), lambda g: (g, 0, 0))],
        out_specs=[pl.BlockSpec((1, _NP, _KP), lambda g: (g, 0, 0))],
        out_shape=[jax.ShapeDtypeStruct((nb, _NP, _KP), jnp.int32)],
        scratch_shapes=[pltpu.VMEM((_NP, _NP), jnp.float32)],
    )(xp)


def _stage_c_body(x_ref, xj_ref, w1a_ref, b1a_ref, w1b_ref, b1b_ref, p1_ref,
                  wg1_ref, hp_ref, idx2_ref, x1_ref, d2_ref, hm_ref):
    g = pl.program_id(0)
    h = jnp.maximum(_conv_max(x_ref[0], xj_ref, w1a_ref, b1a_ref[...],
                              w1b_ref, b1b_ref[...], hm_ref), 0.0)
    p1 = p1_ref[...]
    s = _dotbf_t(h, p1)[:, :1] / (jnp.sqrt(jnp.sum(p1 * p1)) + 1e-16)
    row = lax.broadcasted_iota(jnp.int32, (_NP, 1), 0)
    keep = _select_top(jnp.where(row < _NPG, s, -_BIG), _M1)
    keepf = keep.astype(jnp.float32)
    hp = h * jnp.tanh(s) * keepf
    hp_ref[0] = hp
    x1_ref[0] = _gap(hp, wg1_ref[...], keepf)
    colpen_row = (1.0 - _row_of_col(keepf)) * _BIG
    d2_ref[...] = _pairwise_d2(hp, colpen_row)
    idx_acc = _topk_idx(d2_ref, keep.astype(jnp.int32))
    klane = lax.broadcasted_iota(jnp.int32, (_NP, _KP), 1)
    idx2_ref[0] = jnp.where(klane < _K, idx_acc + g * _NP, idx_acc)


def _run_stage_c(xp, xj1, W1a, b1a_row, W1b, b1b_row, p1_row, wg1_row):
    nb = xp.shape[0]
    return pl.pallas_call(
        _stage_c_body,
        grid=(nb,),
        in_specs=[
            pl.BlockSpec((1, _NP, _F), lambda g: (g, 0, 0)),
            pl.BlockSpec((1, _K * _NP, _F), lambda g: (g, 0, 0)),
            pl.BlockSpec((2 * _F, _H), lambda g: (0, 0)),
            pl.BlockSpec((1, _H), lambda g: (0, 0)),
            pl.BlockSpec((_H, _H), lambda g: (0, 0)),
            pl.BlockSpec((1, _H), lambda g: (0, 0)),
            pl.BlockSpec((8, _H), lambda g: (0, 0)),
            pl.BlockSpec((8, _H), lambda g: (0, 0)),
        ],
        out_specs=[
            pl.BlockSpec((1, _NP, _H), lambda g: (g, 0, 0)),
            pl.BlockSpec((1, _NP, _KP), lambda g: (g, 0, 0)),
            pl.BlockSpec((1, 1, _H), lambda g: (g, 0, 0)),
        ],
        out_shape=[
            jax.ShapeDtypeStruct((nb, _NP, _H), jnp.float32),
            jax.ShapeDtypeStruct((nb, _NP, _KP), jnp.int32),
            jax.ShapeDtypeStruct((nb, 1, _H), jnp.float32),
        ],
        scratch_shapes=[pltpu.VMEM((_NP, _NP), jnp.float32),
                        pltpu.VMEM((_NP, _H), jnp.float32)],
    )(xp, xj1, W1a, b1a_row, W1b, b1b_row, p1_row, wg1_row)


def _stage_e_body(hp_ref, hj_ref, idx2_ref, w2a_ref, b2a_ref, w2b_ref, b2b_ref,
                  p2_ref, wg2_ref, wl_ref, bl_ref, x1_ref, out_ref, hm_ref):
    h2 = _conv_max(hp_ref[0], hj_ref, w2a_ref, b2a_ref[...],
                   w2b_ref, b2b_ref[...], hm_ref)
    keep1 = idx2_ref[0][:, _K:_K + 1] > 0
    p2 = p2_ref[...]
    s2 = _dotbf_t(h2, p2)[:, :1] / (jnp.sqrt(jnp.sum(p2 * p2)) + 1e-16)
    keep2 = _select_top(jnp.where(keep1, s2, -_BIG), _M2)
    keep2f = keep2.astype(jnp.float32)
    hp2 = h2 * jnp.tanh(s2) * keep2f
    x2 = _gap(hp2, wg2_ref[...], keep2f)
    out_ref[0] = _dotbf(x1_ref[0] + x2, wl_ref[...]) + bl_ref[...]


def _run_stage_e(hp, hj2, idx2, W2a, b2a_row, W2b, b2b_row, p2_row, wg2_row,
                 wlp, blp_row, x1):
    nb = hp.shape[0]
    return pl.pallas_call(
        _stage_e_body,
        grid=(nb,),
        in_specs=[
            pl.BlockSpec((1, _NP, _H), lambda g: (g, 0, 0)),
            pl.BlockSpec((1, _K * _NP, _H), lambda g: (g, 0, 0)),
            pl.BlockSpec((1, _NP, _KP), lambda g: (g, 0, 0)),
            pl.BlockSpec((2 * _H, _H), lambda g: (0, 0)),
            pl.BlockSpec((1, _H), lambda g: (0, 0)),
            pl.BlockSpec((_H, _H), lambda g: (0, 0)),
            pl.BlockSpec((1, _H), lambda g: (0, 0)),
            pl.BlockSpec((8, _H), lambda g: (0, 0)),
            pl.BlockSpec((8, _H), lambda g: (0, 0)),
            pl.BlockSpec((_H, _H), lambda g: (0, 0)),
            pl.BlockSpec((1, _H), lambda g: (0, 0)),
            pl.BlockSpec((1, 1, _H), lambda g: (g, 0, 0)),
        ],
        out_specs=[pl.BlockSpec((1, 1, _H), lambda g: (g, 0, 0))],
        out_shape=[jax.ShapeDtypeStruct((nb, 1, _H), jnp.float32)],
        scratch_shapes=[pltpu.VMEM((_NP, _H), jnp.float32)],
    )(hp, hj2, idx2, W2a, b2a_row, W2b, b2b_row, p2_row, wg2_row, wlp, blp_row, x1)


def _sc_gather_rows(table, idx):
    e = idx.shape[0]
    info = plsc.get_sparse_core_info()
    nw = info.num_cores * info.num_subcores
    per_w = e // nw
    nch = -(-per_w // 800)
    while per_w % nch or (per_w // nch) % 8:
        nch += 1
    ch = per_w // nch
    mesh = plsc.VectorSubcoreMesh(core_axis_name="c", subcore_axis_name="s")

    @functools.partial(
        pl.kernel, mesh=mesh,
        out_type=jax.ShapeDtypeStruct((e, _H), jnp.float32),
        scratch_types=[pltpu.VMEM((ch,), jnp.int32),
                       pltpu.VMEM((ch, _H), jnp.float32),
                       pltpu.SemaphoreType.DMA],
    )
    def gk(table_hbm, idx_hbm, out_hbm, idx_v, rows_v, sem):
        wid = lax.axis_index("s") * info.num_cores + lax.axis_index("c")
        base = wid * per_w

        def chunk(c, carry):
            off = pl.multiple_of(base + c * ch, 8)
            pltpu.sync_copy(idx_hbm.at[pl.ds(off, ch)], idx_v)
            pltpu.async_copy(table_hbm.at[idx_v], rows_v, sem).wait()
            pltpu.sync_copy(rows_v, out_hbm.at[pl.ds(off, ch)])
            return carry

        lax.fori_loop(0, nch, chunk, 0)

    return gk(table, idx)


def kernel(x, edge_index, edge_weight, batch, W1a, b1a, W1b, b1b, p1, Wg1, bg1,
           W2a, b2a, W2b, b2b, p2, Wg2, bg2, Wl, bl):
    xp = jnp.pad(x.reshape(_B, _NPG, _F), ((0, 0), (0, _NP - _NPG), (0, 0)))
    wlp = jnp.concatenate([Wl, jnp.zeros((_H, _H - Wl.shape[1]), jnp.float32)], axis=1)
    blp = jnp.concatenate([bl, jnp.zeros((_H - bl.shape[0],), jnp.float32)]).reshape(1, _H)
    p1_8 = jnp.pad(p1.reshape(1, _H), ((0, 7), (0, 0)))
    wg1_8 = jnp.pad(Wg1.reshape(1, _H), ((0, 7), (0, 0)))
    p2_8 = jnp.pad(p2.reshape(1, _H), ((0, 7), (0, 0)))
    wg2_8 = jnp.pad(Wg2.reshape(1, _H), ((0, 7), (0, 0)))

    hb = _B // 2
    outs = []
    for xph in (xp[:hb], xp[hb:]):
        idx1 = _run_stage_a(xph)[0]
        idx1_flat = jnp.transpose(idx1[:, :, :_K], (0, 2, 1)).reshape(-1)
        xj1 = _sc_gather_rows(xph.reshape(hb * _NP, _F), idx1_flat)
        hp, idx2, x1 = _run_stage_c(
            xph, xj1.reshape(hb, _K * _NP, _F), W1a, b1a.reshape(1, _H), W1b,
            b1b.reshape(1, _H), p1_8, wg1_8)
        idx2_flat = jnp.transpose(idx2[:, :, :_K], (0, 2, 1)).reshape(-1)
        hj2 = _sc_gather_rows(hp.reshape(hb * _NP, _H), idx2_flat)
        out = _run_stage_e(hp, hj2.reshape(hb, _K * _NP, _H), idx2, W2a,
                           b2a.reshape(1, _H), W2b, b2b.reshape(1, _H),
                           p2_8, wg2_8, wlp, blp, x1)[0]
        outs.append(out[:, 0, :Wl.shape[1]])
    return jnp.concatenate(outs, axis=0)


# speedup vs baseline: 1.0209x; 1.0099x over previous
import functools

import jax
import jax.numpy as jnp
from jax import lax
from jax.experimental import pallas as pl
from jax.experimental.pallas import tpu as pltpu
from jax.experimental.pallas import tpu_sc as plsc

_B = 20
_NPG = 500
_NP = 512
_F = 128
_H = 128
_K = 20
_KP = 32
_M1 = 400
_M2 = 320
_BIG = 1e30
_JBIG = 2**30


def _dotbf(a, b):
    return lax.dot_general(a.astype(jnp.bfloat16), b.astype(jnp.bfloat16),
                           (((1,), (0,)), ((), ())),
                           preferred_element_type=jnp.float32)


def _dotbf_t(a, b):
    return lax.dot_general(a.astype(jnp.bfloat16), b.astype(jnp.bfloat16),
                           (((1,), (1,)), ((), ())),
                           preferred_element_type=jnp.float32)


def _dot_hi(a, b):
    return lax.dot_general(a, b, (((1,), (0,)), ((), ())),
                           precision=lax.Precision.HIGHEST,
                           preferred_element_type=jnp.float32)


def _dot_hi_t(a, b):
    return lax.dot_general(a, b, (((1,), (1,)), ((), ())),
                           precision=lax.Precision.HIGHEST,
                           preferred_element_type=jnp.float32)


def _pairwise_d2(xp, colpen_row):
    xx = xp * xp
    sq_col = jnp.sum(xx, axis=1, keepdims=True)
    sq_row = _dot_hi_t(jnp.ones((1, _F), jnp.float32), xx)
    mm = _dotbf_t(xp, xp)
    return (sq_col + sq_row) - 2.0 * mm + colpen_row


def _row_of_col(col):
    sub = lax.broadcasted_iota(jnp.int32, (_NP, _NP), 0)
    lan = lax.broadcasted_iota(jnp.int32, (_NP, _NP), 1)
    diag = (sub == lan).astype(jnp.float32) * col
    return _dot_hi(jnp.ones((1, _NP), jnp.float32), diag)


def _topk_idx(d2_ref, keep_i32):
    lane = lax.broadcasted_iota(jnp.int32, (_NP, _NP), 1)
    klane = lax.broadcasted_iota(jnp.int32, (_NP, _KP), 1)

    def body(k, idx_acc):
        d2 = d2_ref[...]
        m = jnp.min(d2, axis=1, keepdims=True)
        cand = jnp.where(d2 == m, lane, _JBIG)
        j = jnp.min(cand, axis=1, keepdims=True)
        idx_acc = jnp.where(klane == k, j, idx_acc)
        d2_ref[...] = jnp.where(lane == j, _BIG, d2)
        return idx_acc

    idx_acc = lax.fori_loop(0, _K, body, jnp.zeros((_NP, _KP), jnp.int32))
    return jnp.where(klane == _K, keep_i32, idx_acc)


def _select_top(svalid, m):
    bits = lax.bitcast_convert_type(svalid, jnp.int32)
    key = jnp.where(bits < 0, bits ^ jnp.int32(0x7FFFFFFF), bits)
    ukey = key ^ jnp.int32(-(2**31))
    hi = lax.shift_right_logical(ukey, jnp.int32(16)).astype(jnp.float32)
    lo = (ukey & jnp.int32(0xFFFF)).astype(jnp.float32)
    hi_row = _row_of_col(hi)
    lo_row = _row_of_col(lo)
    beats = (hi_row > hi) | ((hi_row == hi) & (lo_row > lo))
    sub = lax.broadcasted_iota(jnp.int32, (_NP, _NP), 0)
    lan = lax.broadcasted_iota(jnp.int32, (_NP, _NP), 1)
    tie_before = (hi_row == hi) & (lo_row == lo) & (lan < sub)
    rank = jnp.sum(beats.astype(jnp.float32) + tie_before.astype(jnp.float32),
                   axis=1, keepdims=True)
    return rank < jnp.float32(m)


def _gap(hp, wg_row, keepf):
    gate = _dotbf_t(hp, wg_row)[:, :1]
    gate = jnp.where(keepf > 0.0, gate, -_BIG)
    e = jnp.exp(gate - jnp.max(gate)) * keepf
    w = e / jnp.sum(e)
    return jnp.sum(w * hp, axis=0, keepdims=True)


def _conv_max(xi, xj_ref, wa_ref, ba_row, wb_ref, bb_row, hm_ref):
    hm_ref[...] = jnp.full((_NP, _H), -_BIG, jnp.float32)

    def body(k, carry):
        xj = xj_ref[0, pl.ds(k * _NP, _NP), :]
        z = jnp.concatenate([xi, xj - xi], axis=1)
        msg = _dotbf(jnp.maximum(_dotbf(z, wa_ref[...]) + ba_row, 0.0),
                     wb_ref[...]) + bb_row
        hm_ref[...] = jnp.maximum(hm_ref[...], msg)
        return carry

    lax.fori_loop(0, _K, body, 0)
    return hm_ref[...]


def _stage_a_body(x_ref, idx_ref, d2_ref):
    g = pl.program_id(0)
    x = x_ref[0]
    col = lax.broadcasted_iota(jnp.int32, (1, _NP), 1)
    colpen_row = jnp.where(col >= _NPG, _BIG, 0.0)
    d2_ref[...] = _pairwise_d2(x, colpen_row)
    idx_acc = _topk_idx(d2_ref, jnp.zeros((_NP, 1), jnp.int32))
    row = lax.broadcasted_iota(jnp.int32, (_NP, _KP), 0)
    klane = lax.broadcasted_iota(jnp.int32, (_NP, _KP), 1)
    valid = lax.broadcasted_iota(jnp.int32, (_NP, 1), 0) < _NPG
    idx_acc = jnp.where(valid | (klane >= _K), idx_acc, row)
    idx_ref[0] = jnp.where(klane < _K, idx_acc + g * _NP, idx_acc)


def _run_stage_a(xp):
    nb = xp.shape[0]
    return pl.pallas_call(
        _stage_a_body,
        grid=(nb,),
        in_specs=[pl.BlockSpec((1, _NP, _F), lambda g: (g, 0, 0))],
        out_specs=[pl.BlockSpec((1, _NP, _KP), lambda g: (g, 0, 0))],
        out_shape=[jax.ShapeDtypeStruct((nb, _NP, _KP), jnp.int32)],
        scratch_shapes=[pltpu.VMEM((_NP, _NP), jnp.float32)],
    )(xp)


def _stage_c_body(x_ref, xj_ref, w1a_ref, b1a_ref, w1b_ref, b1b_ref, p1_ref,
                  wg1_ref, hp_ref, idx2_ref, x1_ref, d2_ref, hm_ref):
    g = pl.program_id(0)
    h = jnp.maximum(_conv_max(x_ref[0], xj_ref, w1a_ref, b1a_ref[...],
                              w1b_ref, b1b_ref[...], hm_ref), 0.0)
    p1 = p1_ref[...]
    s = _dotbf_t(h, p1)[:, :1] / (jnp.sqrt(jnp.sum(p1 * p1)) + 1e-16)
    row = lax.broadcasted_iota(jnp.int32, (_NP, 1), 0)
    keep = _select_top(jnp.where(row < _NPG, s, -_BIG), _M1)
    keepf = keep.astype(jnp.float32)
    hp = h * jnp.tanh(s) * keepf
    hp_ref[0] = hp
    x1_ref[0] = _gap(hp, wg1_ref[...], keepf)
    colpen_row = (1.0 - _row_of_col(keepf)) * _BIG
    d2_ref[...] = _pairwise_d2(hp, colpen_row)
    idx_acc = _topk_idx(d2_ref, keep.astype(jnp.int32))
    row = lax.broadcasted_iota(jnp.int32, (_NP, _KP), 0)
    klane = lax.broadcasted_iota(jnp.int32, (_NP, _KP), 1)
    idx_acc = jnp.where(keep | (klane >= _K), idx_acc, row)
    idx2_ref[0] = jnp.where(klane < _K, idx_acc + g * _NP, idx_acc)


def _run_stage_c(xp, xj1, W1a, b1a_row, W1b, b1b_row, p1_row, wg1_row):
    nb = xp.shape[0]
    return pl.pallas_call(
        _stage_c_body,
        grid=(nb,),
        in_specs=[
            pl.BlockSpec((1, _NP, _F), lambda g: (g, 0, 0)),
            pl.BlockSpec((1, _K * _NP, _F), lambda g: (g, 0, 0)),
            pl.BlockSpec((2 * _F, _H), lambda g: (0, 0)),
            pl.BlockSpec((1, _H), lambda g: (0, 0)),
            pl.BlockSpec((_H, _H), lambda g: (0, 0)),
            pl.BlockSpec((1, _H), lambda g: (0, 0)),
            pl.BlockSpec((8, _H), lambda g: (0, 0)),
            pl.BlockSpec((8, _H), lambda g: (0, 0)),
        ],
        out_specs=[
            pl.BlockSpec((1, _NP, _H), lambda g: (g, 0, 0)),
            pl.BlockSpec((1, _NP, _KP), lambda g: (g, 0, 0)),
            pl.BlockSpec((1, 1, _H), lambda g: (g, 0, 0)),
        ],
        out_shape=[
            jax.ShapeDtypeStruct((nb, _NP, _H), jnp.float32),
            jax.ShapeDtypeStruct((nb, _NP, _KP), jnp.int32),
            jax.ShapeDtypeStruct((nb, 1, _H), jnp.float32),
        ],
        scratch_shapes=[pltpu.VMEM((_NP, _NP), jnp.float32),
                        pltpu.VMEM((_NP, _H), jnp.float32)],
    )(xp, xj1, W1a, b1a_row, W1b, b1b_row, p1_row, wg1_row)


def _stage_e_body(hp_ref, hj_ref, idx2_ref, w2a_ref, b2a_ref, w2b_ref, b2b_ref,
                  p2_ref, wg2_ref, wl_ref, bl_ref, x1_ref, out_ref, hm_ref):
    h2 = _conv_max(hp_ref[0], hj_ref, w2a_ref, b2a_ref[...],
                   w2b_ref, b2b_ref[...], hm_ref)
    keep1 = idx2_ref[0][:, _K:_K + 1] > 0
    p2 = p2_ref[...]
    s2 = _dotbf_t(h2, p2)[:, :1] / (jnp.sqrt(jnp.sum(p2 * p2)) + 1e-16)
    keep2 = _select_top(jnp.where(keep1, s2, -_BIG), _M2)
    keep2f = keep2.astype(jnp.float32)
    hp2 = h2 * jnp.tanh(s2) * keep2f
    x2 = _gap(hp2, wg2_ref[...], keep2f)
    out_ref[0] = _dotbf(x1_ref[0] + x2, wl_ref[...]) + bl_ref[...]


def _run_stage_e(hp, hj2, idx2, W2a, b2a_row, W2b, b2b_row, p2_row, wg2_row,
                 wlp, blp_row, x1):
    nb = hp.shape[0]
    return pl.pallas_call(
        _stage_e_body,
        grid=(nb,),
        in_specs=[
            pl.BlockSpec((1, _NP, _H), lambda g: (g, 0, 0)),
            pl.BlockSpec((1, _K * _NP, _H), lambda g: (g, 0, 0)),
            pl.BlockSpec((1, _NP, _KP), lambda g: (g, 0, 0)),
            pl.BlockSpec((2 * _H, _H), lambda g: (0, 0)),
            pl.BlockSpec((1, _H), lambda g: (0, 0)),
            pl.BlockSpec((_H, _H), lambda g: (0, 0)),
            pl.BlockSpec((1, _H), lambda g: (0, 0)),
            pl.BlockSpec((8, _H), lambda g: (0, 0)),
            pl.BlockSpec((8, _H), lambda g: (0, 0)),
            pl.BlockSpec((_H, _H), lambda g: (0, 0)),
            pl.BlockSpec((1, _H), lambda g: (0, 0)),
            pl.BlockSpec((1, 1, _H), lambda g: (g, 0, 0)),
        ],
        out_specs=[pl.BlockSpec((1, 1, _H), lambda g: (g, 0, 0))],
        out_shape=[jax.ShapeDtypeStruct((nb, 1, _H), jnp.float32)],
        scratch_shapes=[pltpu.VMEM((_NP, _H), jnp.float32)],
    )(hp, hj2, idx2, W2a, b2a_row, W2b, b2b_row, p2_row, wg2_row, wlp, blp_row, x1)


def _sc_gather_rows(table, idx):
    e = idx.shape[0]
    info = plsc.get_sparse_core_info()
    nw = info.num_cores * info.num_subcores
    per_w = e // nw
    nch = -(-per_w // 400)
    while per_w % nch or nch % 2 or (per_w // nch) % 8:
        nch += 1
    ch = per_w // nch
    mesh = plsc.VectorSubcoreMesh(core_axis_name="c", subcore_axis_name="s")

    @functools.partial(
        pl.kernel, mesh=mesh,
        out_type=jax.ShapeDtypeStruct((e, _H), jnp.float32),
        scratch_types=[pltpu.VMEM((ch,), jnp.int32),
                       pltpu.VMEM((ch,), jnp.int32),
                       pltpu.VMEM((ch, _H), jnp.float32),
                       pltpu.VMEM((ch, _H), jnp.float32),
                       pltpu.SemaphoreType.DMA,
                       pltpu.SemaphoreType.DMA],
    )
    def gk(table_hbm, idx_hbm, out_hbm, idx_v0, idx_v1, rows_v0, rows_v1,
           sem0, sem1):
        wid = lax.axis_index("s") * info.num_cores + lax.axis_index("c")
        base = wid * per_w
        bufs = ((idx_v0, rows_v0, sem0), (idx_v1, rows_v1, sem1))

        def start(c, b):
            idx_v, rows_v, sem = bufs[b]
            off = pl.multiple_of(base + c * ch, 8)
            pltpu.sync_copy(idx_hbm.at[pl.ds(off, ch)], idx_v)
            pltpu.async_copy(table_hbm.at[idx_v], rows_v, sem)

        start(0, 0)
        start(1, 1)

        def body(c2, carry):
            for b in (0, 1):
                idx_v, rows_v, sem = bufs[b]
                c = c2 * 2 + b
                pltpu.make_async_copy(table_hbm.at[idx_v], rows_v, sem).wait()
                off = pl.multiple_of(base + c * ch, 8)
                pltpu.sync_copy(rows_v, out_hbm.at[pl.ds(off, ch)])

                @pl.when(c + 2 < nch)
                def _():
                    start(c + 2, b)
            return carry

        lax.fori_loop(0, nch // 2, body, 0)

    return gk(table, idx)


def kernel(x, edge_index, edge_weight, batch, W1a, b1a, W1b, b1b, p1, Wg1, bg1,
           W2a, b2a, W2b, b2b, p2, Wg2, bg2, Wl, bl):
    xp = jnp.pad(x.reshape(_B, _NPG, _F), ((0, 0), (0, _NP - _NPG), (0, 0)))
    wlp = jnp.concatenate([Wl, jnp.zeros((_H, _H - Wl.shape[1]), jnp.float32)], axis=1)
    blp = jnp.concatenate([bl, jnp.zeros((_H - bl.shape[0],), jnp.float32)]).reshape(1, _H)
    p1_8 = jnp.pad(p1.reshape(1, _H), ((0, 7), (0, 0)))
    wg1_8 = jnp.pad(Wg1.reshape(1, _H), ((0, 7), (0, 0)))
    p2_8 = jnp.pad(p2.reshape(1, _H), ((0, 7), (0, 0)))
    wg2_8 = jnp.pad(Wg2.reshape(1, _H), ((0, 7), (0, 0)))

    hb = _B // 2
    outs = []
    for xph in (xp[:hb], xp[hb:]):
        idx1 = _run_stage_a(xph)[0]
        idx1_flat = jnp.transpose(idx1[:, :, :_K], (0, 2, 1)).reshape(-1)
        xj1 = _sc_gather_rows(xph.reshape(hb * _NP, _F), idx1_flat)
        hp, idx2, x1 = _run_stage_c(
            xph, xj1.reshape(hb, _K * _NP, _F), W1a, b1a.reshape(1, _H), W1b,
            b1b.reshape(1, _H), p1_8, wg1_8)
        idx2_flat = jnp.transpose(idx2[:, :, :_K], (0, 2, 1)).reshape(-1)
        hj2 = _sc_gather_rows(hp.reshape(hb * _NP, _H), idx2_flat)
        out = _run_stage_e(hp, hj2.reshape(hb, _K * _NP, _H), idx2, W2a,
                           b2a.reshape(1, _H), W2b, b2b.reshape(1, _H),
                           p2_8, wg2_8, wlp, blp, x1)[0]
        outs.append(out[:, 0, :Wl.shape[1]])
    return jnp.concatenate(outs, axis=0)


# speedup vs baseline: 1.0387x; 1.0175x over previous
import functools

import jax
import jax.numpy as jnp
from jax import lax
from jax.experimental import pallas as pl
from jax.experimental.pallas import tpu as pltpu
from jax.experimental.pallas import tpu_sc as plsc

_B = 20
_NPG = 500
_NP = 512
_F = 128
_H = 128
_K = 20
_KP = 32
_M1 = 400
_M2 = 320
_BIG = 1e30
_JBIG = 2**30


def _dotbf(a, b):
    return lax.dot_general(a.astype(jnp.bfloat16), b.astype(jnp.bfloat16),
                           (((1,), (0,)), ((), ())),
                           preferred_element_type=jnp.float32)


def _dotbf_t(a, b):
    return lax.dot_general(a.astype(jnp.bfloat16), b.astype(jnp.bfloat16),
                           (((1,), (1,)), ((), ())),
                           preferred_element_type=jnp.float32)


def _dot_hi(a, b):
    return lax.dot_general(a, b, (((1,), (0,)), ((), ())),
                           precision=lax.Precision.HIGHEST,
                           preferred_element_type=jnp.float32)


def _dot_hi_t(a, b):
    return lax.dot_general(a, b, (((1,), (1,)), ((), ())),
                           precision=lax.Precision.HIGHEST,
                           preferred_element_type=jnp.float32)


def _pairwise_d2(xp, colpen_col):
    xx = xp * xp
    sq_col = jnp.sum(xx, axis=1, keepdims=True)
    sq_row = _dot_hi_t(jnp.ones((1, _F), jnp.float32), xx)
    mm = _dotbf_t(xp, xp)
    return (sq_col + sq_row) - 2.0 * mm + colpen_col


def _row_of_col(col):
    sub = lax.broadcasted_iota(jnp.int32, (_NP, _NP), 0)
    lan = lax.broadcasted_iota(jnp.int32, (_NP, _NP), 1)
    diag = (sub == lan).astype(jnp.float32) * col
    return _dot_hi(jnp.ones((1, _NP), jnp.float32), diag)


def _col_of_row(row):
    sub = lax.broadcasted_iota(jnp.int32, (_NP, _NP), 0)
    lan = lax.broadcasted_iota(jnp.int32, (_NP, _NP), 1)
    diag = (sub == lan).astype(jnp.float32) * row
    return _dot_hi(diag, jnp.ones((_NP, 1), jnp.float32))


def _topk_idx(d2_ref, keep_row_i32):
    sub = lax.broadcasted_iota(jnp.int32, (_NP, _NP), 0)
    ksub = lax.broadcasted_iota(jnp.int32, (_KP, _NP), 0)

    def body(k, idx_acc):
        d2 = d2_ref[...]
        m = jnp.min(d2, axis=0, keepdims=True)
        cand = jnp.where(d2 == m, sub, _JBIG)
        j = jnp.min(cand, axis=0, keepdims=True)
        idx_acc = jnp.where(ksub == k, j, idx_acc)
        d2_ref[...] = jnp.where(sub == j, _BIG, d2)
        return idx_acc

    idx_acc = lax.fori_loop(0, _K, body, jnp.zeros((_KP, _NP), jnp.int32))
    return jnp.where(ksub == _K, keep_row_i32, idx_acc)


def _select_top(svalid, m):
    bits = lax.bitcast_convert_type(svalid, jnp.int32)
    key = jnp.where(bits < 0, bits ^ jnp.int32(0x7FFFFFFF), bits)
    ukey = key ^ jnp.int32(-(2**31))
    hi = lax.shift_right_logical(ukey, jnp.int32(16)).astype(jnp.float32)
    lo = (ukey & jnp.int32(0xFFFF)).astype(jnp.float32)
    hi_row = _row_of_col(hi)
    lo_row = _row_of_col(lo)
    beats = (hi_row > hi) | ((hi_row == hi) & (lo_row > lo))
    sub = lax.broadcasted_iota(jnp.int32, (_NP, _NP), 0)
    lan = lax.broadcasted_iota(jnp.int32, (_NP, _NP), 1)
    tie_before = (hi_row == hi) & (lo_row == lo) & (lan < sub)
    rank = jnp.sum(beats.astype(jnp.float32) + tie_before.astype(jnp.float32),
                   axis=1, keepdims=True)
    return rank < jnp.float32(m)


def _gap(hp, wg_row, keepf):
    gate = _dotbf_t(hp, wg_row)[:, :1]
    gate = jnp.where(keepf > 0.0, gate, -_BIG)
    e = jnp.exp(gate - jnp.max(gate)) * keepf
    w = e / jnp.sum(e)
    return jnp.sum(w * hp, axis=0, keepdims=True)


def _conv_max(xi, xj_ref, wa_ref, ba_row, wb_ref, bb_row, hm_ref):
    hm_ref[...] = jnp.full((_NP, _H), -_BIG, jnp.float32)

    def body(k, carry):
        xj = xj_ref[0, pl.ds(k * _NP, _NP), :]
        z = jnp.concatenate([xi, xj - xi], axis=1)
        msg = _dotbf(jnp.maximum(_dotbf(z, wa_ref[...]) + ba_row, 0.0),
                     wb_ref[...]) + bb_row
        hm_ref[...] = jnp.maximum(hm_ref[...], msg)
        return carry

    lax.fori_loop(0, _K, body, 0)
    return hm_ref[...]


def _stage_a_body(x_ref, idx_ref, d2_ref):
    g = pl.program_id(0)
    x = x_ref[0]
    sub1 = lax.broadcasted_iota(jnp.int32, (_NP, 1), 0)
    colpen_col = jnp.where(sub1 >= _NPG, _BIG, 0.0)
    d2_ref[...] = _pairwise_d2(x, colpen_col)
    idx_acc = _topk_idx(d2_ref, jnp.zeros((1, _NP), jnp.int32))
    ksub = lax.broadcasted_iota(jnp.int32, (_KP, _NP), 0)
    node = lax.broadcasted_iota(jnp.int32, (_KP, _NP), 1)
    valid = lax.broadcasted_iota(jnp.int32, (1, _NP), 1) < _NPG
    idx_acc = jnp.where(valid | (ksub >= _K), idx_acc, node)
    idx_ref[0] = jnp.where(ksub < _K, idx_acc + g * _NP, idx_acc)


def _run_stage_a(xp):
    nb = xp.shape[0]
    return pl.pallas_call(
        _stage_a_body,
        grid=(nb,),
        in_specs=[pl.BlockSpec((1, _NP, _F), lambda g: (g, 0, 0))],
        out_specs=[pl.BlockSpec((1, _KP, _NP), lambda g: (g, 0, 0))],
        out_shape=[jax.ShapeDtypeStruct((nb, _KP, _NP), jnp.int32)],
        scratch_shapes=[pltpu.VMEM((_NP, _NP), jnp.float32)],
    )(xp)


def _stage_c_body(x_ref, xj_ref, w1a_ref, b1a_ref, w1b_ref, b1b_ref, p1_ref,
                  wg1_ref, hp_ref, idx2_ref, x1_ref, d2_ref, hm_ref):
    g = pl.program_id(0)
    h = jnp.maximum(_conv_max(x_ref[0], xj_ref, w1a_ref, b1a_ref[...],
                              w1b_ref, b1b_ref[...], hm_ref), 0.0)
    p1 = p1_ref[...]
    s = _dotbf_t(h, p1)[:, :1] / (jnp.sqrt(jnp.sum(p1 * p1)) + 1e-16)
    row = lax.broadcasted_iota(jnp.int32, (_NP, 1), 0)
    keep = _select_top(jnp.where(row < _NPG, s, -_BIG), _M1)
    keepf = keep.astype(jnp.float32)
    hp = h * jnp.tanh(s) * keepf
    hp_ref[0] = hp
    x1_ref[0] = _gap(hp, wg1_ref[...], keepf)
    colpen_col = (1.0 - keepf) * _BIG
    d2_ref[...] = _pairwise_d2(hp, colpen_col)
    keep_row = _row_of_col(keepf)
    idx_acc = _topk_idx(d2_ref, keep_row.astype(jnp.int32))
    ksub = lax.broadcasted_iota(jnp.int32, (_KP, _NP), 0)
    node = lax.broadcasted_iota(jnp.int32, (_KP, _NP), 1)
    idx_acc = jnp.where((keep_row > 0.5) | (ksub >= _K), idx_acc, node)
    idx2_ref[0] = jnp.where(ksub < _K, idx_acc + g * _NP, idx_acc)


def _run_stage_c(xp, xj1, W1a, b1a_row, W1b, b1b_row, p1_row, wg1_row):
    nb = xp.shape[0]
    return pl.pallas_call(
        _stage_c_body,
        grid=(nb,),
        in_specs=[
            pl.BlockSpec((1, _NP, _F), lambda g: (g, 0, 0)),
            pl.BlockSpec((1, _K * _NP, _F), lambda g: (g, 0, 0)),
            pl.BlockSpec((2 * _F, _H), lambda g: (0, 0)),
            pl.BlockSpec((1, _H), lambda g: (0, 0)),
            pl.BlockSpec((_H, _H), lambda g: (0, 0)),
            pl.BlockSpec((1, _H), lambda g: (0, 0)),
            pl.BlockSpec((8, _H), lambda g: (0, 0)),
            pl.BlockSpec((8, _H), lambda g: (0, 0)),
        ],
        out_specs=[
            pl.BlockSpec((1, _NP, _H), lambda g: (g, 0, 0)),
            pl.BlockSpec((1, _KP, _NP), lambda g: (g, 0, 0)),
            pl.BlockSpec((1, 1, _H), lambda g: (g, 0, 0)),
        ],
        out_shape=[
            jax.ShapeDtypeStruct((nb, _NP, _H), jnp.float32),
            jax.ShapeDtypeStruct((nb, _KP, _NP), jnp.int32),
            jax.ShapeDtypeStruct((nb, 1, _H), jnp.float32),
        ],
        scratch_shapes=[pltpu.VMEM((_NP, _NP), jnp.float32),
                        pltpu.VMEM((_NP, _H), jnp.float32)],
    )(xp, xj1, W1a, b1a_row, W1b, b1b_row, p1_row, wg1_row)


def _stage_e_body(hp_ref, hj_ref, idx2_ref, w2a_ref, b2a_ref, w2b_ref, b2b_ref,
                  p2_ref, wg2_ref, wl_ref, bl_ref, x1_ref, out_ref, hm_ref):
    h2 = _conv_max(hp_ref[0], hj_ref, w2a_ref, b2a_ref[...],
                   w2b_ref, b2b_ref[...], hm_ref)
    keep1_row = idx2_ref[0][_K:_K + 1, :].astype(jnp.float32)
    keep1 = _col_of_row(keep1_row) > 0.5
    p2 = p2_ref[...]
    s2 = _dotbf_t(h2, p2)[:, :1] / (jnp.sqrt(jnp.sum(p2 * p2)) + 1e-16)
    keep2 = _select_top(jnp.where(keep1, s2, -_BIG), _M2)
    keep2f = keep2.astype(jnp.float32)
    hp2 = h2 * jnp.tanh(s2) * keep2f
    x2 = _gap(hp2, wg2_ref[...], keep2f)
    out_ref[0] = _dotbf(x1_ref[0] + x2, wl_ref[...]) + bl_ref[...]


def _run_stage_e(hp, hj2, idx2, W2a, b2a_row, W2b, b2b_row, p2_row, wg2_row,
                 wlp, blp_row, x1):
    nb = hp.shape[0]
    return pl.pallas_call(
        _stage_e_body,
        grid=(nb,),
        in_specs=[
            pl.BlockSpec((1, _NP, _H), lambda g: (g, 0, 0)),
            pl.BlockSpec((1, _K * _NP, _H), lambda g: (g, 0, 0)),
            pl.BlockSpec((1, _KP, _NP), lambda g: (g, 0, 0)),
            pl.BlockSpec((2 * _H, _H), lambda g: (0, 0)),
            pl.BlockSpec((1, _H), lambda g: (0, 0)),
            pl.BlockSpec((_H, _H), lambda g: (0, 0)),
            pl.BlockSpec((1, _H), lambda g: (0, 0)),
            pl.BlockSpec((8, _H), lambda g: (0, 0)),
            pl.BlockSpec((8, _H), lambda g: (0, 0)),
            pl.BlockSpec((_H, _H), lambda g: (0, 0)),
            pl.BlockSpec((1, _H), lambda g: (0, 0)),
            pl.BlockSpec((1, 1, _H), lambda g: (g, 0, 0)),
        ],
        out_specs=[pl.BlockSpec((1, 1, _H), lambda g: (g, 0, 0))],
        out_shape=[jax.ShapeDtypeStruct((nb, 1, _H), jnp.float32)],
        scratch_shapes=[pltpu.VMEM((_NP, _H), jnp.float32)],
    )(hp, hj2, idx2, W2a, b2a_row, W2b, b2b_row, p2_row, wg2_row, wlp, blp_row, x1)


def _sc_gather_rows(table, idx):
    e = idx.shape[0]
    info = plsc.get_sparse_core_info()
    nw = info.num_cores * info.num_subcores
    per_w = e // nw
    nch = -(-per_w // 400)
    while per_w % nch or nch % 2 or (per_w // nch) % 8:
        nch += 1
    ch = per_w // nch
    mesh = plsc.VectorSubcoreMesh(core_axis_name="c", subcore_axis_name="s")

    @functools.partial(
        pl.kernel, mesh=mesh,
        out_type=jax.ShapeDtypeStruct((e, _H), jnp.float32),
        scratch_types=[pltpu.VMEM((ch,), jnp.int32),
                       pltpu.VMEM((ch,), jnp.int32),
                       pltpu.VMEM((ch, _H), jnp.float32),
                       pltpu.VMEM((ch, _H), jnp.float32),
                       pltpu.SemaphoreType.DMA,
                       pltpu.SemaphoreType.DMA],
    )
    def gk(table_hbm, idx_hbm, out_hbm, idx_v0, idx_v1, rows_v0, rows_v1,
           sem0, sem1):
        wid = lax.axis_index("s") * info.num_cores + lax.axis_index("c")
        base = wid * per_w
        bufs = ((idx_v0, rows_v0, sem0), (idx_v1, rows_v1, sem1))

        def start(c, b):
            idx_v, rows_v, sem = bufs[b]
            off = pl.multiple_of(base + c * ch, 8)
            pltpu.sync_copy(idx_hbm.at[pl.ds(off, ch)], idx_v)
            pltpu.async_copy(table_hbm.at[idx_v], rows_v, sem)

        start(0, 0)
        start(1, 1)

        def body(c2, carry):
            for b in (0, 1):
                idx_v, rows_v, sem = bufs[b]
                c = c2 * 2 + b
                pltpu.make_async_copy(table_hbm.at[idx_v], rows_v, sem).wait()
                off = pl.multiple_of(base + c * ch, 8)
                pltpu.sync_copy(rows_v, out_hbm.at[pl.ds(off, ch)])

                @pl.when(c + 2 < nch)
                def _():
                    start(c + 2, b)
            return carry

        lax.fori_loop(0, nch // 2, body, 0)

    return gk(table, idx)


def kernel(x, edge_index, edge_weight, batch, W1a, b1a, W1b, b1b, p1, Wg1, bg1,
           W2a, b2a, W2b, b2b, p2, Wg2, bg2, Wl, bl):
    xp = jnp.pad(x.reshape(_B, _NPG, _F), ((0, 0), (0, _NP - _NPG), (0, 0)))
    wlp = jnp.concatenate([Wl, jnp.zeros((_H, _H - Wl.shape[1]), jnp.float32)], axis=1)
    blp = jnp.concatenate([bl, jnp.zeros((_H - bl.shape[0],), jnp.float32)]).reshape(1, _H)
    p1_8 = jnp.pad(p1.reshape(1, _H), ((0, 7), (0, 0)))
    wg1_8 = jnp.pad(Wg1.reshape(1, _H), ((0, 7), (0, 0)))
    p2_8 = jnp.pad(p2.reshape(1, _H), ((0, 7), (0, 0)))
    wg2_8 = jnp.pad(Wg2.reshape(1, _H), ((0, 7), (0, 0)))

    hb = _B // 2
    outs = []
    for xph in (xp[:hb], xp[hb:]):
        idx1 = _run_stage_a(xph)[0]
        idx1_flat = idx1[:, :_K, :].reshape(-1)
        xj1 = _sc_gather_rows(xph.reshape(hb * _NP, _F), idx1_flat)
        hp, idx2, x1 = _run_stage_c(
            xph, xj1.reshape(hb, _K * _NP, _F), W1a, b1a.reshape(1, _H), W1b,
            b1b.reshape(1, _H), p1_8, wg1_8)
        idx2_flat = idx2[:, :_K, :].reshape(-1)
        hj2 = _sc_gather_rows(hp.reshape(hb * _NP, _H), idx2_flat)
        out = _run_stage_e(hp, hj2.reshape(hb, _K * _NP, _H), idx2, W2a,
                           b2a.reshape(1, _H), W2b, b2b.reshape(1, _H),
                           p2_8, wg2_8, wlp, blp, x1)[0]
        outs.append(out[:, 0, :Wl.shape[1]])
    return jnp.concatenate(outs, axis=0)


# speedup vs baseline: 1.1484x; 1.1055x over previous
import functools

import jax
import jax.numpy as jnp
from jax import lax
from jax.experimental import pallas as pl
from jax.experimental.pallas import tpu as pltpu
from jax.experimental.pallas import tpu_sc as plsc

_B = 20
_NPG = 500
_NP = 512
_F = 128
_H = 128
_K = 20
_KP = 32
_M1 = 400
_M2 = 320
_BIG = 1e30
_JBIG = 2**30


def _dotbf(a, b):
    return lax.dot_general(a.astype(jnp.bfloat16), b.astype(jnp.bfloat16),
                           (((1,), (0,)), ((), ())),
                           preferred_element_type=jnp.float32)


def _dotbf_t(a, b):
    return lax.dot_general(a.astype(jnp.bfloat16), b.astype(jnp.bfloat16),
                           (((1,), (1,)), ((), ())),
                           preferred_element_type=jnp.float32)


def _dot_hi(a, b):
    return lax.dot_general(a, b, (((1,), (0,)), ((), ())),
                           precision=lax.Precision.HIGHEST,
                           preferred_element_type=jnp.float32)


def _dot_hi_t(a, b):
    return lax.dot_general(a, b, (((1,), (1,)), ((), ())),
                           precision=lax.Precision.HIGHEST,
                           preferred_element_type=jnp.float32)


def _pairwise_d2(xp, colpen_col):
    xx = xp * xp
    sq_col = jnp.sum(xx, axis=1, keepdims=True)
    sq_row = _dot_hi_t(jnp.ones((1, _F), jnp.float32), xx)
    mm = _dotbf_t(xp, xp)
    return (sq_col + sq_row) - 2.0 * mm + colpen_col


def _row_of_col(col):
    sub = lax.broadcasted_iota(jnp.int32, (_NP, _NP), 0)
    lan = lax.broadcasted_iota(jnp.int32, (_NP, _NP), 1)
    diag = (sub == lan).astype(jnp.float32) * col
    return _dot_hi(jnp.ones((1, _NP), jnp.float32), diag)


def _col_of_row(row):
    sub = lax.broadcasted_iota(jnp.int32, (_NP, _NP), 0)
    lan = lax.broadcasted_iota(jnp.int32, (_NP, _NP), 1)
    diag = (sub == lan).astype(jnp.float32) * row
    return _dot_hi(diag, jnp.ones((_NP, 1), jnp.float32))


def _topk_idx(d2_ref, keep_row_i32):
    sub = lax.broadcasted_iota(jnp.int32, (_NP, _NP), 0)
    ksub = lax.broadcasted_iota(jnp.int32, (_KP, _NP), 0)

    def body(k, idx_acc):
        d2 = d2_ref[...]
        m = jnp.min(d2, axis=0, keepdims=True)
        cand = jnp.where(d2 == m, sub, _JBIG)
        j = jnp.min(cand, axis=0, keepdims=True)
        idx_acc = jnp.where(ksub == k, j, idx_acc)
        d2_ref[...] = jnp.where(sub == j, _BIG, d2)
        return idx_acc

    idx_acc = lax.fori_loop(0, _K, body, jnp.zeros((_KP, _NP), jnp.int32))
    return jnp.where(ksub == _K, keep_row_i32, idx_acc)


def _select_top(svalid, m):
    bits = lax.bitcast_convert_type(svalid, jnp.int32)
    key = jnp.where(bits < 0, bits ^ jnp.int32(0x7FFFFFFF), bits)
    ukey = key ^ jnp.int32(-(2**31))
    hi = lax.shift_right_logical(ukey, jnp.int32(16)).astype(jnp.float32)
    lo = (ukey & jnp.int32(0xFFFF)).astype(jnp.float32)
    hi_row = _row_of_col(hi)
    lo_row = _row_of_col(lo)
    beats = (hi_row > hi) | ((hi_row == hi) & (lo_row > lo))
    sub = lax.broadcasted_iota(jnp.int32, (_NP, _NP), 0)
    lan = lax.broadcasted_iota(jnp.int32, (_NP, _NP), 1)
    tie_before = (hi_row == hi) & (lo_row == lo) & (lan < sub)
    rank = jnp.sum(beats.astype(jnp.float32) + tie_before.astype(jnp.float32),
                   axis=1, keepdims=True)
    return rank < jnp.float32(m)


def _gap(hp, wg_row, keepf):
    gate = _dotbf_t(hp, wg_row)[:, :1]
    gate = jnp.where(keepf > 0.0, gate, -_BIG)
    e = jnp.exp(gate - jnp.max(gate)) * keepf
    w = e / jnp.sum(e)
    return jnp.sum(w * hp, axis=0, keepdims=True)


def _conv_max(xi, xj_ref, wa_ref, ba_row, wb_ref, bb_row, hm_ref):
    u = _dotbf(xi, wa_ref[:_F, :]) + ba_row
    hm_ref[...] = jnp.full((_NP, _H), -_BIG, jnp.float32)

    def body(k, carry):
        xj = xj_ref[0, pl.ds(k * _NP, _NP), :]
        msg = _dotbf(jnp.maximum(u + _dotbf(xj - xi, wa_ref[_F:, :]), 0.0),
                     wb_ref[...]) + bb_row
        hm_ref[...] = jnp.maximum(hm_ref[...], msg)
        return carry

    lax.fori_loop(0, _K, body, 0)
    return hm_ref[...]


def _stage_a_body(x_ref, idx_ref, d2_ref):
    g = pl.program_id(0)
    x = x_ref[0]
    sub1 = lax.broadcasted_iota(jnp.int32, (_NP, 1), 0)
    colpen_col = jnp.where(sub1 >= _NPG, _BIG, 0.0)
    d2_ref[...] = _pairwise_d2(x, colpen_col)
    idx_acc = _topk_idx(d2_ref, jnp.zeros((1, _NP), jnp.int32))
    ksub = lax.broadcasted_iota(jnp.int32, (_KP, _NP), 0)
    node = lax.broadcasted_iota(jnp.int32, (_KP, _NP), 1)
    valid = lax.broadcasted_iota(jnp.int32, (1, _NP), 1) < _NPG
    idx_acc = jnp.where(valid | (ksub >= _K), idx_acc, node)
    idx_ref[0] = jnp.where(ksub < _K, idx_acc + g * _NP, idx_acc)


def _run_stage_a(xp):
    nb = xp.shape[0]
    return pl.pallas_call(
        _stage_a_body,
        grid=(nb,),
        in_specs=[pl.BlockSpec((1, _NP, _F), lambda g: (g, 0, 0))],
        out_specs=[pl.BlockSpec((1, _KP, _NP), lambda g: (g, 0, 0))],
        out_shape=[jax.ShapeDtypeStruct((nb, _KP, _NP), jnp.int32)],
        scratch_shapes=[pltpu.VMEM((_NP, _NP), jnp.float32)],
    )(xp)


def _stage_c_body(x_ref, xj_ref, w1a_ref, b1a_ref, w1b_ref, b1b_ref, p1_ref,
                  wg1_ref, hp_ref, idx2_ref, x1_ref, d2_ref, hm_ref):
    g = pl.program_id(0)
    h = jnp.maximum(_conv_max(x_ref[0], xj_ref, w1a_ref, b1a_ref[...],
                              w1b_ref, b1b_ref[...], hm_ref), 0.0)
    p1 = p1_ref[...]
    s = _dotbf_t(h, p1)[:, :1] / (jnp.sqrt(jnp.sum(p1 * p1)) + 1e-16)
    row = lax.broadcasted_iota(jnp.int32, (_NP, 1), 0)
    keep = _select_top(jnp.where(row < _NPG, s, -_BIG), _M1)
    keepf = keep.astype(jnp.float32)
    hp = h * jnp.tanh(s) * keepf
    hp_ref[0] = hp
    x1_ref[0] = _gap(hp, wg1_ref[...], keepf)
    colpen_col = (1.0 - keepf) * _BIG
    d2_ref[...] = _pairwise_d2(hp, colpen_col)
    keep_row = _row_of_col(keepf)
    idx_acc = _topk_idx(d2_ref, keep_row.astype(jnp.int32))
    ksub = lax.broadcasted_iota(jnp.int32, (_KP, _NP), 0)
    node = lax.broadcasted_iota(jnp.int32, (_KP, _NP), 1)
    idx_acc = jnp.where((keep_row > 0.5) | (ksub >= _K), idx_acc, node)
    idx2_ref[0] = jnp.where(ksub < _K, idx_acc + g * _NP, idx_acc)


def _run_stage_c(xp, xj1, W1a, b1a_row, W1b, b1b_row, p1_row, wg1_row):
    nb = xp.shape[0]
    return pl.pallas_call(
        _stage_c_body,
        grid=(nb,),
        in_specs=[
            pl.BlockSpec((1, _NP, _F), lambda g: (g, 0, 0)),
            pl.BlockSpec((1, _K * _NP, _F), lambda g: (g, 0, 0)),
            pl.BlockSpec((2 * _F, _H), lambda g: (0, 0)),
            pl.BlockSpec((1, _H), lambda g: (0, 0)),
            pl.BlockSpec((_H, _H), lambda g: (0, 0)),
            pl.BlockSpec((1, _H), lambda g: (0, 0)),
            pl.BlockSpec((8, _H), lambda g: (0, 0)),
            pl.BlockSpec((8, _H), lambda g: (0, 0)),
        ],
        out_specs=[
            pl.BlockSpec((1, _NP, _H), lambda g: (g, 0, 0)),
            pl.BlockSpec((1, _KP, _NP), lambda g: (g, 0, 0)),
            pl.BlockSpec((1, 1, _H), lambda g: (g, 0, 0)),
        ],
        out_shape=[
            jax.ShapeDtypeStruct((nb, _NP, _H), jnp.float32),
            jax.ShapeDtypeStruct((nb, _KP, _NP), jnp.int32),
            jax.ShapeDtypeStruct((nb, 1, _H), jnp.float32),
        ],
        scratch_shapes=[pltpu.VMEM((_NP, _NP), jnp.float32),
                        pltpu.VMEM((_NP, _H), jnp.float32)],
    )(xp, xj1, W1a, b1a_row, W1b, b1b_row, p1_row, wg1_row)


def _stage_e_body(hp_ref, hj_ref, idx2_ref, w2a_ref, b2a_ref, w2b_ref, b2b_ref,
                  p2_ref, wg2_ref, wl_ref, bl_ref, x1_ref, out_ref, hm_ref):
    h2 = _conv_max(hp_ref[0], hj_ref, w2a_ref, b2a_ref[...],
                   w2b_ref, b2b_ref[...], hm_ref)
    keep1_row = idx2_ref[0][_K:_K + 1, :].astype(jnp.float32)
    keep1 = _col_of_row(keep1_row) > 0.5
    p2 = p2_ref[...]
    s2 = _dotbf_t(h2, p2)[:, :1] / (jnp.sqrt(jnp.sum(p2 * p2)) + 1e-16)
    keep2 = _select_top(jnp.where(keep1, s2, -_BIG), _M2)
    keep2f = keep2.astype(jnp.float32)
    hp2 = h2 * jnp.tanh(s2) * keep2f
    x2 = _gap(hp2, wg2_ref[...], keep2f)
    out_ref[0] = _dotbf(x1_ref[0] + x2, wl_ref[...]) + bl_ref[...]


def _run_stage_e(hp, hj2, idx2, W2a, b2a_row, W2b, b2b_row, p2_row, wg2_row,
                 wlp, blp_row, x1):
    nb = hp.shape[0]
    return pl.pallas_call(
        _stage_e_body,
        grid=(nb,),
        in_specs=[
            pl.BlockSpec((1, _NP, _H), lambda g: (g, 0, 0)),
            pl.BlockSpec((1, _K * _NP, _H), lambda g: (g, 0, 0)),
            pl.BlockSpec((1, _KP, _NP), lambda g: (g, 0, 0)),
            pl.BlockSpec((2 * _H, _H), lambda g: (0, 0)),
            pl.BlockSpec((1, _H), lambda g: (0, 0)),
            pl.BlockSpec((_H, _H), lambda g: (0, 0)),
            pl.BlockSpec((1, _H), lambda g: (0, 0)),
            pl.BlockSpec((8, _H), lambda g: (0, 0)),
            pl.BlockSpec((8, _H), lambda g: (0, 0)),
            pl.BlockSpec((_H, _H), lambda g: (0, 0)),
            pl.BlockSpec((1, _H), lambda g: (0, 0)),
            pl.BlockSpec((1, 1, _H), lambda g: (g, 0, 0)),
        ],
        out_specs=[pl.BlockSpec((1, 1, _H), lambda g: (g, 0, 0))],
        out_shape=[jax.ShapeDtypeStruct((nb, 1, _H), jnp.float32)],
        scratch_shapes=[pltpu.VMEM((_NP, _H), jnp.float32)],
    )(hp, hj2, idx2, W2a, b2a_row, W2b, b2b_row, p2_row, wg2_row, wlp, blp_row, x1)


def _sc_gather_rows(table, idx):
    e = idx.shape[0]
    info = plsc.get_sparse_core_info()
    nw = info.num_cores * info.num_subcores
    per_w = e // nw
    nch = -(-per_w // 400)
    while per_w % nch or nch % 2 or (per_w // nch) % 8:
        nch += 1
    ch = per_w // nch
    mesh = plsc.VectorSubcoreMesh(core_axis_name="c", subcore_axis_name="s")

    @functools.partial(
        pl.kernel, mesh=mesh,
        out_type=jax.ShapeDtypeStruct((e, _H), jnp.float32),
        scratch_types=[pltpu.VMEM((ch,), jnp.int32),
                       pltpu.VMEM((ch,), jnp.int32),
                       pltpu.VMEM((ch, _H), jnp.float32),
                       pltpu.VMEM((ch, _H), jnp.float32),
                       pltpu.SemaphoreType.DMA,
                       pltpu.SemaphoreType.DMA],
    )
    def gk(table_hbm, idx_hbm, out_hbm, idx_v0, idx_v1, rows_v0, rows_v1,
           sem0, sem1):
        wid = lax.axis_index("s") * info.num_cores + lax.axis_index("c")
        base = wid * per_w
        bufs = ((idx_v0, rows_v0, sem0), (idx_v1, rows_v1, sem1))

        def start(c, b):
            idx_v, rows_v, sem = bufs[b]
            off = pl.multiple_of(base + c * ch, 8)
            pltpu.sync_copy(idx_hbm.at[pl.ds(off, ch)], idx_v)
            pltpu.async_copy(table_hbm.at[idx_v], rows_v, sem)

        start(0, 0)
        start(1, 1)

        def body(c2, carry):
            for b in (0, 1):
                idx_v, rows_v, sem = bufs[b]
                c = c2 * 2 + b
                pltpu.make_async_copy(table_hbm.at[idx_v], rows_v, sem).wait()
                off = pl.multiple_of(base + c * ch, 8)
                pltpu.sync_copy(rows_v, out_hbm.at[pl.ds(off, ch)])

                @pl.when(c + 2 < nch)
                def _():
                    start(c + 2, b)
            return carry

        lax.fori_loop(0, nch // 2, body, 0)

    return gk(table, idx)


def kernel(x, edge_index, edge_weight, batch, W1a, b1a, W1b, b1b, p1, Wg1, bg1,
           W2a, b2a, W2b, b2b, p2, Wg2, bg2, Wl, bl):
    xp = jnp.pad(x.reshape(_B, _NPG, _F), ((0, 0), (0, _NP - _NPG), (0, 0)))
    wlp = jnp.concatenate([Wl, jnp.zeros((_H, _H - Wl.shape[1]), jnp.float32)], axis=1)
    blp = jnp.concatenate([bl, jnp.zeros((_H - bl.shape[0],), jnp.float32)]).reshape(1, _H)
    p1_8 = jnp.pad(p1.reshape(1, _H), ((0, 7), (0, 0)))
    wg1_8 = jnp.pad(Wg1.reshape(1, _H), ((0, 7), (0, 0)))
    p2_8 = jnp.pad(p2.reshape(1, _H), ((0, 7), (0, 0)))
    wg2_8 = jnp.pad(Wg2.reshape(1, _H), ((0, 7), (0, 0)))

    hb = _B // 2
    outs = []
    for xph in (xp[:hb], xp[hb:]):
        idx1 = _run_stage_a(xph)[0]
        idx1_flat = idx1[:, :_K, :].reshape(-1)
        xj1 = _sc_gather_rows(xph.reshape(hb * _NP, _F), idx1_flat)
        hp, idx2, x1 = _run_stage_c(
            xph, xj1.reshape(hb, _K * _NP, _F), W1a, b1a.reshape(1, _H), W1b,
            b1b.reshape(1, _H), p1_8, wg1_8)
        idx2_flat = idx2[:, :_K, :].reshape(-1)
        hj2 = _sc_gather_rows(hp.reshape(hb * _NP, _H), idx2_flat)
        out = _run_stage_e(hp, hj2.reshape(hb, _K * _NP, _H), idx2, W2a,
                           b2a.reshape(1, _H), W2b, b2b.reshape(1, _H),
                           p2_8, wg2_8, wlp, blp, x1)[0]
        outs.append(out[:, 0, :Wl.shape[1]])
    return jnp.concatenate(outs, axis=0)
